# Initial kernel scaffold; baseline (speedup 1.0000x reference)
import jax
import jax.numpy as jnp
from jax.experimental import pallas as pl


def kernel(obs, lstm_h, lstm_c, W_ih, W_hh, b, W_a, b_a, W_v, b_v):
    raise NotImplementedError("write your pallas kernel here")


# speedup vs baseline: 1.9151x; 1.9151x over previous
import functools

import jax
import jax.numpy as jnp
from jax.experimental import pallas as pl

P = 8
N = 16384
D = 32
H = 32
A = 8
Q = N // P
GATES = 4 * P * H
BQ = 512


def _lstm_pool_body(x_ref, h_ref, c_ref, wcat_ref, b_ref, wav_ref, bav_ref,
                    act_ref, lgp_ref, ent_ref, val_ref, ho_ref, co_ref):
    x = x_ref[...]
    h = h_ref[...]
    c = c_ref[...]
    xh = jnp.concatenate([x, h], axis=1)
    gates = jnp.dot(xh, wcat_ref[...],
                    preferred_element_type=jnp.float32) + b_ref[...]
    PH = P * H
    ig = jax.nn.sigmoid(gates[:, 0 * PH:1 * PH])
    fg = jax.nn.sigmoid(gates[:, 1 * PH:2 * PH])
    gg = jnp.tanh(gates[:, 2 * PH:3 * PH])
    og = jax.nn.sigmoid(gates[:, 3 * PH:4 * PH])
    c_new = fg * c + ig * gg
    h_new = og * jnp.tanh(c_new)
    ho_ref[...] = h_new
    co_ref[...] = c_new

    for p in range(P):
        hp = h_new[:, p * H:(p + 1) * H]
        la = jnp.dot(hp, wav_ref[p].T,
                     preferred_element_type=jnp.float32) + bav_ref[p:p + 1, :]
        logits = la[:, :A]
        val = la[:, A:A + 1]
        m = jnp.max(logits, axis=-1, keepdims=True)
        ex = jnp.exp(logits - m)
        s = jnp.sum(ex, axis=-1, keepdims=True)
        logp = (logits - m) - jnp.log(s)
        probs = ex / s
        idx = jax.lax.broadcasted_iota(jnp.int32, (logits.shape[0], A), 1)
        atn = jnp.min(jnp.where(logits == m, idx, A), axis=-1, keepdims=True)
        lgp = jnp.max(logp, axis=-1, keepdims=True)
        ent = -jnp.sum(probs * logp, axis=-1, keepdims=True)
        act_ref[:, p:p + 1] = atn
        lgp_ref[:, p:p + 1] = lgp
        ent_ref[:, p:p + 1] = ent
        val_ref[:, p:p + 1] = val


@functools.partial(jax.jit, static_argnames=("interpret",))
def _run(obs, lstm_h, lstm_c, W_ih, W_hh, b, W_a, b_a, W_v, b_v,
         interpret=False):
    eye = jnp.eye(P, dtype=jnp.float32)
    w4 = W_ih.reshape(P, 4, H, D)
    wih_big = jnp.einsum('pgjd,pq->pdgqj', w4, eye).reshape(P * D, GATES)
    w4h = W_hh.reshape(P, 4, H, H)
    whh_big = jnp.einsum('pgjd,pq->pdgqj', w4h, eye).reshape(P * H, GATES)
    wcat = jnp.concatenate([wih_big, whh_big], axis=0)
    b_big = jnp.transpose(b.reshape(P, 4, H), (1, 0, 2)).reshape(1, GATES)
    wav = jnp.zeros((P, 16, H), jnp.float32)
    wav = wav.at[:, :A, :].set(W_a).at[:, A, :].set(W_v[:, 0, :])
    bav = jnp.zeros((P, 16), jnp.float32)
    bav = bav.at[:, :A].set(b_a).at[:, A].set(b_v[:, 0])

    x2 = obs.reshape(Q, P * D)
    h2 = lstm_h.reshape(Q, P * H)
    c2 = lstm_c.reshape(Q, P * H)

    grid = (Q // BQ,)
    row_spec = lambda w: pl.BlockSpec((BQ, w), lambda i: (i, 0))
    full = lambda shape: pl.BlockSpec(shape, lambda i: (0,) * len(shape))

    out_shapes = (
        jax.ShapeDtypeStruct((Q, P), jnp.int32),
        jax.ShapeDtypeStruct((Q, P), jnp.float32),
        jax.ShapeDtypeStruct((Q, P), jnp.float32),
        jax.ShapeDtypeStruct((Q, P), jnp.float32),
        jax.ShapeDtypeStruct((Q, P * H), jnp.float32),
        jax.ShapeDtypeStruct((Q, P * H), jnp.float32),
    )
    act, lgp, ent, val, ho, co = pl.pallas_call(
        _lstm_pool_body,
        grid=grid,
        in_specs=[
            row_spec(P * D), row_spec(P * H), row_spec(P * H),
            full((P * (D + H), GATES)), full((1, GATES)),
            full((P, 16, H)), full((P, 16)),
        ],
        out_specs=[
            row_spec(P), row_spec(P), row_spec(P), row_spec(P),
            row_spec(P * H), row_spec(P * H),
        ],
        out_shape=out_shapes,
        interpret=interpret,
    )(x2, h2, c2, wcat, b_big, wav, bav)

    return (act.reshape(N), lgp.reshape(N), ent.reshape(N), val.reshape(N),
            (ho.reshape(1, N, H), co.reshape(1, N, H)))


def kernel(obs, lstm_h, lstm_c, W_ih, W_hh, b, W_a, b_a, W_v, b_v):
    return _run(obs, lstm_h, lstm_c, W_ih, W_hh, b, W_a, b_a, W_v, b_v)
